# baseline (speedup 1.0000x reference)
import functools
import numpy as np
import jax
import jax.numpy as jnp
from jax.experimental import pallas as pl

L = 12
DIM = 2
T = 2 ** 19
BASE = 16
DESIRED = 2048
PRIMES = (1, 2654435761, 805459861)

_B = np.exp((np.log(DESIRED) - np.log(BASE)) / (L - 1))
_RES = tuple(int(np.floor(BASE * (_B ** l))) for l in range(L))
_CORNERS = tuple((dx, dy, dz) for dx in (0, 1) for dy in (0, 1) for dz in (0, 1))

_S = 2048


def _core_body(frac_ref, ts_ref, te_ref, rows_d_ref, rows_c_ref,
               w1d_ref, w2d_ref, w1c_ref, w2c_ref,
               sdt_ref, alpha_ref, rgb_ref):
    fr = [frac_ref[:, a, :] for a in range(3)]
    ii = jax.lax.broadcasted_iota(jnp.int32, (L, 2 * L), 0)
    jj = jax.lax.broadcasted_iota(jnp.int32, (L, 2 * L), 1)
    E = (jj // 2 == ii).astype(jnp.float32)
    f24 = [jnp.dot(f, E, preferred_element_type=jnp.float32) for f in fr]

    feat_d = jnp.zeros((fr[0].shape[0], 2 * L), jnp.float32)
    feat_c = jnp.zeros((fr[0].shape[0], 2 * L), jnp.float32)
    for c, (dx, dy, dz) in enumerate(_CORNERS):
        wx = f24[0] if dx else 1.0 - f24[0]
        wy = f24[1] if dy else 1.0 - f24[1]
        wz = f24[2] if dz else 1.0 - f24[2]
        w = wx * wy * wz
        feat_d = feat_d + w * rows_d_ref[c]
        feat_c = feat_c + w * rows_c_ref[c]

    hd = jnp.maximum(jnp.dot(feat_d, w1d_ref[...],
                             preferred_element_type=jnp.float32), 0.0)
    sig = jnp.exp(jnp.clip(jnp.dot(hd, w2d_ref[...],
                                   preferred_element_type=jnp.float32),
                           -15.0, 15.0))[:, 0]
    hc = jnp.maximum(jnp.dot(feat_c, w1c_ref[...],
                             preferred_element_type=jnp.float32), 0.0)
    logits = jnp.dot(hc, w2c_ref[...], preferred_element_type=jnp.float32)
    rgb_ref[...] = 1.0 / (1.0 + jnp.exp(-logits))

    sdt = sig * (te_ref[...] - ts_ref[...])
    sdt_ref[...] = sdt
    alpha_ref[...] = 1.0 - jnp.exp(-sdt)


def kernel(rays_o, rays_d, t_starts, t_ends, ray_indices, table_d, table_c,
           w1_d, w2_d, w1_c, w2_c):
    n_rays = rays_o.shape[0]
    n = t_starts.shape[0]
    t_mid = (t_starts + t_ends)[:, None] * 0.5
    xs = rays_o[ray_indices] + rays_d[ray_indices] * t_mid
    xs_n = (jnp.clip(xs, -1.0, 1.0) + 1.0) / 2.0

    res_f = jnp.asarray(_RES, jnp.float32)[None, :, None]
    pos = xs_n[:, None, :] * res_f
    p0f = jnp.floor(pos)
    p0 = p0f.astype(jnp.uint32)
    frac = jnp.transpose(pos - p0f, (0, 2, 1))
    lofs = (jnp.arange(L, dtype=jnp.int32) * T)[None, :]
    td = table_d.reshape(L * T, DIM)
    tc = table_c.reshape(L * T, DIM)
    rows_d, rows_c = [], []
    for dx, dy, dz in _CORNERS:
        cx = p0[:, :, 0] + np.uint32(dx)
        cy = p0[:, :, 1] + np.uint32(dy)
        cz = p0[:, :, 2] + np.uint32(dz)
        h = (cx * np.uint32(PRIMES[0])) ^ (cy * np.uint32(PRIMES[1])) \
            ^ (cz * np.uint32(PRIMES[2]))
        idx = (h & np.uint32(T - 1)).astype(jnp.int32) + lofs
        rows_d.append(td[idx].reshape(n, 2 * L))
        rows_c.append(tc[idx].reshape(n, 2 * L))
    rows_d = jnp.stack(rows_d)
    rows_c = jnp.stack(rows_c)

    grid = (n // _S,)
    sdt, alphas, rgbs = pl.pallas_call(
        _core_body,
        grid=grid,
        in_specs=[
            pl.BlockSpec((_S, 3, L), lambda i: (i, 0, 0)),
            pl.BlockSpec((_S,), lambda i: (i,)),
            pl.BlockSpec((_S,), lambda i: (i,)),
            pl.BlockSpec((8, _S, 2 * L), lambda i: (0, i, 0)),
            pl.BlockSpec((8, _S, 2 * L), lambda i: (0, i, 0)),
            pl.BlockSpec((2 * L, 32), lambda i: (0, 0)),
            pl.BlockSpec((32, 1), lambda i: (0, 0)),
            pl.BlockSpec((2 * L, 32), lambda i: (0, 0)),
            pl.BlockSpec((32, 3), lambda i: (0, 0)),
        ],
        out_specs=[
            pl.BlockSpec((_S,), lambda i: (i,)),
            pl.BlockSpec((_S,), lambda i: (i,)),
            pl.BlockSpec((_S, 3), lambda i: (i, 0)),
        ],
        out_shape=[
            jax.ShapeDtypeStruct((n,), jnp.float32),
            jax.ShapeDtypeStruct((n,), jnp.float32),
            jax.ShapeDtypeStruct((n, 3), jnp.float32),
        ],
    )(frac, t_starts, t_ends, rows_d, rows_c, w1_d, w2_d, w1_c, w2_c)

    cum = jnp.concatenate([jnp.zeros((1,), jnp.float32), jnp.cumsum(sdt)])
    first_idx = jnp.searchsorted(ray_indices,
                                 jnp.arange(n_rays, dtype=ray_indices.dtype))
    offsets = cum[first_idx]
    within_excl = cum[:-1] - offsets[ray_indices]
    trans = jnp.exp(-within_excl)
    weights = trans * alphas
    color = jax.ops.segment_sum(weights[:, None] * rgbs, ray_indices,
                                num_segments=n_rays)
    acc = jax.ops.segment_sum(weights, ray_indices, num_segments=n_rays)
    color = jnp.clip(color + (1.0 - acc[:, None]), 0.0, 1.0)
    alpha = jnp.clip(acc, 0.0, 1.0)
    return (color, alpha)
